# baseline (speedup 1.0000x reference)
import functools

import jax
import jax.numpy as jnp
from jax import lax
from jax.experimental import pallas as pl
from jax.experimental.pallas import tpu as pltpu
from jax.experimental.pallas import tpu_sc as plsc

_SEG = 128
_NSEG = 26
_F = _SEG * _NSEG
_B = 16384
_NW = 32
_ROWS = _B // _NW


def _permute_body(in_hbm, out_hbm, sem):
    wid = lax.axis_index("s") * 2 + lax.axis_index("c")
    base = wid * _ROWS
    copies = []
    for j in range(_NSEG):
        src_col = (_NSEG - 1 - j) * _SEG
        copies.append(pltpu.make_async_copy(
            in_hbm.at[pl.ds(base, _ROWS), pl.ds(src_col, _SEG)],
            out_hbm.at[pl.ds(base, _ROWS), pl.ds(j * _SEG, _SEG)],
            sem,
        ))
    for c in copies:
        c.start()
    for c in copies:
        c.wait()


@jax.jit
def kernel(pooled_embs):
    run = pl.kernel(
        _permute_body,
        out_type=jax.ShapeDtypeStruct((_B, _F), jnp.float32),
        mesh=plsc.VectorSubcoreMesh(core_axis_name="c", subcore_axis_name="s"),
        scratch_types=[pltpu.SemaphoreType.DMA],
    )
    return run(pooled_embs)
